# Initial kernel scaffold; baseline (speedup 1.0000x reference)
import jax
import jax.numpy as jnp
from jax.experimental import pallas as pl


def kernel(x, my_input_1, conv_mask_W, Wr, Wi):
    raise NotImplementedError("write your pallas kernel here")


# speedup vs baseline: 3.4203x; 3.4203x over previous
import jax
import jax.numpy as jnp
from jax.experimental import pallas as pl
from jax.experimental.pallas import tpu as pltpu

B, NCOILS = 16, 8
H = W = 256
MH = MW = 64
KH = KW = 3
NSTACK = 2
K_PER = 512
NCH = 2 * NCOILS
NTAP = KH * KW
KDIM = NCH * NTAP
HCHUNK = 64
HI_BITS = 0x3F800001


def _net_kernel(x2_ref, wmat_ref, xr_ref, xi_ref, out_ref, xp_ref, xq_ref):
    x2 = x2_ref[0]
    bits = jax.lax.bitcast_convert_type(x2, jnp.int32)

    def bisect(_, lohi):
        lo, hi = lohi
        mid = jax.lax.div(lo + hi, jnp.int32(2))
        cnt = jnp.sum((bits >= mid).astype(jnp.int32))
        big = cnt >= K_PER
        return (jnp.where(big, mid, lo), jnp.where(big, hi, mid))

    lo, _ = jax.lax.fori_loop(0, 31, bisect,
                              (jnp.int32(0), jnp.int32(HI_BITS)))
    mask64 = jnp.logical_and(bits >= lo, x2 > 0.0).astype(jnp.float32)
    mask = jnp.tile(mask64, (H // MH, W // MW))

    xr = xr_ref[0]
    xi = xi_ref[0]
    m3 = mask[None, :, :]

    @pl.when(pl.program_id(0) == 0)
    def _():
        xp_ref[...] = jnp.zeros_like(xp_ref)
        xq_ref[...] = jnp.zeros_like(xq_ref)

    xp_ref[:NCOILS, 1:H + 1, 1:W + 1] = xr * m3
    xp_ref[NCOILS:, 1:H + 1, 1:W + 1] = xi * m3

    for s in range(NSTACK):
        src = xp_ref if s == 0 else xq_ref
        wmat = wmat_ref[s]
        for h0 in range(0, H, HCHUNK):
            parts = []
            for dy in range(KH):
                for dx in range(KW):
                    sl = src[:, h0 + dy:h0 + dy + HCHUNK, dx:dx + W]
                    parts.append(sl.reshape(NCH, HCHUNK * W))
            a = jnp.concatenate(parts, axis=0)
            o = jax.lax.dot_general(
                wmat, a, (((1,), (0,)), ((), ())),
                preferred_element_type=jnp.float32)
            o = o.reshape(NCH, HCHUNK, W)
            mch = m3[:, h0:h0 + HCHUNK, :]
            xch = jnp.concatenate(
                [xr[:, h0:h0 + HCHUNK, :], xi[:, h0:h0 + HCHUNK, :]], axis=0)
            pred = jnp.where(mch > 0.5, xch, o)
            if s == 0:
                xq_ref[:, 1 + h0:1 + h0 + HCHUNK, 1:W + 1] = pred
            else:
                out_ref[0, :, h0:h0 + HCHUNK, :] = pred


@jax.jit
def kernel(x, my_input_1, conv_mask_W, Wr, Wi):
    xr = x[..., 0]
    xi = x[..., 1]
    conv_out = jax.lax.conv_transpose(
        my_input_1, conv_mask_W, strides=(1, 1), padding='VALID',
        dimension_numbers=('NCHW', 'IOHW', 'NCHW'), transpose_kernel=True)
    x2 = jax.nn.sigmoid(conv_out).reshape(B, MH, MW)

    wr0 = Wr.at[..., KH // 2, KW // 2].set(0.0)
    wi0 = Wi.at[..., KH // 2, KW // 2].set(0.0)
    top = jnp.concatenate([wr0, -wi0], axis=2)
    bot = jnp.concatenate([wi0, wr0], axis=2)
    mfull = jnp.concatenate([top, bot], axis=1)
    wmat = mfull.transpose(0, 1, 3, 4, 2).reshape(NSTACK, NCH, KDIM)

    out16 = pl.pallas_call(
        _net_kernel,
        grid=(B,),
        in_specs=[
            pl.BlockSpec((1, MH, MW), lambda b: (b, 0, 0)),
            pl.BlockSpec((NSTACK, NCH, KDIM), lambda b: (0, 0, 0)),
            pl.BlockSpec((1, NCOILS, H, W), lambda b: (b, 0, 0, 0)),
            pl.BlockSpec((1, NCOILS, H, W), lambda b: (b, 0, 0, 0)),
        ],
        out_specs=pl.BlockSpec((1, NCH, H, W), lambda b: (b, 0, 0, 0)),
        out_shape=jax.ShapeDtypeStruct((B, NCH, H, W), jnp.float32),
        scratch_shapes=[
            pltpu.VMEM((NCH, H + 2, W + 2), jnp.float32),
            pltpu.VMEM((NCH, H + 2, W + 2), jnp.float32),
        ],
    )(x2, wmat, xr, xi)

    return jnp.stack((out16[:, :NCOILS], out16[:, NCOILS:]), axis=-1)
